# baseline (speedup 1.0000x reference)
import functools

import jax
import jax.numpy as jnp
from jax import lax
from jax.experimental import pallas as pl
from jax.experimental.pallas import tpu as pltpu
from jax.experimental.pallas import tpu_sc as plsc


def kernel(x, table):
    B = x.shape[0]
    V, D = table.shape

    info = plsc.get_sparse_core_info()
    NW = info.num_subcores
    assert B % NW == 0
    b_per_w = B // NW

    mesh = plsc.VectorSubcoreMesh(
        core_axis_name="c", subcore_axis_name="s", num_cores=1
    )

    @functools.partial(
        pl.kernel,
        mesh=mesh,
        out_type=jax.ShapeDtypeStruct((B, D), jnp.float32),
        scratch_types=[
            pltpu.VMEM((b_per_w,), jnp.int32),
            pltpu.VMEM((b_per_w, D), jnp.float32),
            pltpu.VMEM_SHARED((V, D), jnp.float32),
            pltpu.SemaphoreType.DMA,
            pltpu.SemaphoreType.DMA,
        ],
        compiler_params=pltpu.CompilerParams(
            use_tc_tiling_on_sc=False,
            skip_device_barrier=True,
        ),
    )
    def gather_kernel(
        table_hbm, idx_hbm, out_hbm, idx_v, rows_v, tab_sp, sem_g, sem_s
    ):
        wid = lax.axis_index("s")
        base = wid * b_per_w
        half = b_per_w // 2

        @pl.when(wid == 0)
        def _():
            pltpu.sync_copy(table_hbm, tab_sp)

        pltpu.sync_copy(idx_hbm.at[pl.ds(base, b_per_w)], idx_v)
        plsc.subcore_barrier()
        nchunk = 8
        c = b_per_w // nchunk
        gathers = [
            pltpu.async_copy(
                tab_sp.at[idx_v.at[pl.ds(k * c, c)]],
                rows_v.at[pl.ds(k * c, c)], sem_g)
            for k in range(nchunk)
        ]
        stores = []
        for k in range(nchunk):
            gathers[k].wait()
            stores.append(
                pltpu.async_copy(
                    rows_v.at[pl.ds(k * c, c)],
                    out_hbm.at[pl.ds(base + k * c, c)], sem_s))
        for s in stores:
            s.wait()

    return gather_kernel(table, x.astype(jnp.int32))


# speedup vs baseline: 1.0044x; 1.0044x over previous
import functools

import jax
import jax.numpy as jnp
from jax import lax
from jax.experimental import pallas as pl
from jax.experimental.pallas import tpu as pltpu
from jax.experimental.pallas import tpu_sc as plsc


def kernel(x, table):
    B = x.shape[0]
    V, D = table.shape

    info = plsc.get_sparse_core_info()
    NC, NS = info.num_cores, info.num_subcores
    NW = NC * NS
    assert B % NW == 0
    b_per_w = B // NW

    mesh = plsc.VectorSubcoreMesh(core_axis_name="c", subcore_axis_name="s")

    @functools.partial(
        pl.kernel,
        mesh=mesh,
        out_type=jax.ShapeDtypeStruct((B, D), jnp.float32),
        scratch_types=[
            pltpu.VMEM((b_per_w,), jnp.int32),
            pltpu.VMEM((b_per_w, D), jnp.float32),
            pltpu.VMEM_SHARED((V, D), jnp.float32),
            pltpu.SemaphoreType.DMA,
            pltpu.SemaphoreType.DMA,
        ],
        compiler_params=pltpu.CompilerParams(
            use_tc_tiling_on_sc=False,
            skip_device_barrier=True,
        ),
    )
    def gather_kernel(
        table_hbm, idx_hbm, out_hbm, idx_v, rows_v, tab_sp, sem_g, sem_s
    ):
        sid = lax.axis_index("s")
        wid = sid * NC + lax.axis_index("c")
        base = wid * b_per_w

        @pl.when(sid == 0)
        def _():
            pltpu.sync_copy(table_hbm, tab_sp)

        pltpu.sync_copy(idx_hbm.at[pl.ds(base, b_per_w)], idx_v)
        plsc.subcore_barrier()
        nchunk = 4
        c = b_per_w // nchunk
        gathers = [
            pltpu.async_copy(
                tab_sp.at[idx_v.at[pl.ds(k * c, c)]],
                rows_v.at[pl.ds(k * c, c)], sem_g)
            for k in range(nchunk)
        ]
        stores = []
        for k in range(nchunk):
            gathers[k].wait()
            stores.append(
                pltpu.async_copy(
                    rows_v.at[pl.ds(k * c, c)],
                    out_hbm.at[pl.ds(base + k * c, c)], sem_s))
        for s in stores:
            s.wait()

    return gather_kernel(table, x.astype(jnp.int32))
